# Initial kernel scaffold; baseline (speedup 1.0000x reference)
import jax
import jax.numpy as jnp
from jax.experimental import pallas as pl


def kernel(x, edge_index, edge_weight, W_in, b_in, convW, convB, temp, feW1, feB1, feW2, feB2, W_out, b_out):
    raise NotImplementedError("write your pallas kernel here")


# speedup vs baseline: 2.6430x; 2.6430x over previous
import jax
import jax.numpy as jnp
from jax import lax
from jax.experimental import pallas as pl
from jax.experimental.pallas import tpu as pltpu
from jax.experimental.pallas import tpu_sc as plsc

_NC = 2
_NS = 16
_LN = 16
_C = 128

_BN = 1000
_BE = 2528


def _mesh():
    return plsc.VectorSubcoreMesh(core_axis_name="c", subcore_axis_name="s",
                                  num_cores=_NC, num_subcores=_NS)


def _edge_pass(hh, srcp, dstp, wexp, zeros, n, ch):
    nps = (n // _NS) // 8 * 8
    tail = n - _NS * nps

    def body(hh_ref, src_ref, dst_ref, wx_ref, z_ref, out_ref,
             sidx_v, didx_v, w_v, rows_v, acc_sh, gsem):
        c = lax.axis_index("c")
        s = lax.axis_index("s")
        wid = c * _NS + s

        pltpu.sync_copy(z_ref.at[pl.ds(s * nps, nps)],
                        acc_sh.at[pl.ds(s * nps, nps)])
        if tail:
            @pl.when(s == _NS - 1)
            def _():
                pltpu.sync_copy(z_ref.at[pl.ds(_NS * nps, tail)],
                                acc_sh.at[pl.ds(_NS * nps, tail)])
        plsc.subcore_barrier()

        def chunk(j, carry):
            off = (wid * ch + j) * _C
            pltpu.sync_copy(src_ref.at[pl.ds(off, _C)], sidx_v)
            pltpu.sync_copy(dst_ref.at[pl.ds(off, _C)], didx_v)
            pltpu.sync_copy(wx_ref.at[pl.ds(off, _C)], w_v)
            pltpu.async_copy(hh_ref.at[sidx_v], rows_v, gsem).wait()

            def edge(e, cc):
                ws = w_v[e, pl.ds(0, _LN)]
                for k in range(8):
                    sl = rows_v[e, pl.ds(k * _LN, _LN)]
                    rows_v[e, pl.ds(k * _LN, _LN)] = sl * ws
                return cc

            lax.fori_loop(0, _C, edge, 0)
            pltpu.sync_copy(rows_v, acc_sh.at[didx_v], add=True)
            return carry

        lax.fori_loop(0, ch, chunk, 0)
        plsc.subcore_barrier()

        pltpu.sync_copy(acc_sh.at[pl.ds(s * nps, nps)],
                        out_ref.at[c, pl.ds(s * nps, nps)])
        if tail:
            @pl.when(s == _NS - 1)
            def _():
                pltpu.sync_copy(acc_sh.at[pl.ds(_NS * nps, tail)],
                                out_ref.at[c, pl.ds(_NS * nps, tail)])

    f = pl.kernel(
        body,
        out_type=jax.ShapeDtypeStruct((_NC, n, 128), jnp.float32),
        mesh=_mesh(),
        scratch_types=[
            pltpu.VMEM((_C,), jnp.int32),
            pltpu.VMEM((_C,), jnp.int32),
            pltpu.VMEM((_C, _LN), jnp.float32),
            pltpu.VMEM((_C, 128), jnp.float32),
            pltpu.VMEM_SHARED((n, 128), jnp.float32),
            pltpu.SemaphoreType.DMA,
        ],
    )
    return f(hh, srcp, dstp, wexp, zeros)


def _edge_attn(g, nrm, srcp, dstp, epad, ch):
    def body(g_ref, nrm_ref, src_ref, dst_ref, pb_ref, den_ref,
             sidx_v, didx_v, rows1_v, rows2_v, ns_v, nd_v, pbuf_v, den_v,
             sem1, sem2, sem3, sem4):
        c = lax.axis_index("c")
        s = lax.axis_index("s")
        wid = c * _NS + s

        def chunk(j, carry):
            off = (wid * ch + j) * _C
            pltpu.sync_copy(src_ref.at[pl.ds(off, _C)], sidx_v)
            pltpu.sync_copy(dst_ref.at[pl.ds(off, _C)], didx_v)
            cp1 = pltpu.async_copy(g_ref.at[sidx_v], rows1_v, sem1)
            cp2 = pltpu.async_copy(g_ref.at[didx_v], rows2_v, sem2)
            cp3 = pltpu.async_copy(nrm_ref.at[sidx_v], ns_v, sem3)
            cp4 = pltpu.async_copy(nrm_ref.at[didx_v], nd_v, sem4)
            cp1.wait()
            cp2.wait()
            cp3.wait()
            cp4.wait()

            def edge(e, cc):
                acc = rows1_v[e, pl.ds(0, _LN)] * rows2_v[e, pl.ds(0, _LN)]
                for k in range(1, 8):
                    acc = acc + (rows1_v[e, pl.ds(k * _LN, _LN)] *
                                 rows2_v[e, pl.ds(k * _LN, _LN)])
                pbuf_v[e, pl.ds(0, _LN)] = acc
                return cc

            lax.fori_loop(0, _C, edge, 0)

            for gi in range(8):
                sl = pl.ds(gi * _LN, _LN)
                den_v[sl] = jnp.maximum(ns_v[sl] * nd_v[sl],
                                        jnp.float32(1e-8))

            pltpu.sync_copy(pbuf_v, pb_ref.at[pl.ds(off, _C)])
            pltpu.sync_copy(den_v, den_ref.at[pl.ds(off, _C)])
            return carry

        lax.fori_loop(0, ch, chunk, 0)

    f = pl.kernel(
        body,
        out_type=[jax.ShapeDtypeStruct((epad, _LN), jnp.float32),
                  jax.ShapeDtypeStruct((epad,), jnp.float32)],
        mesh=_mesh(),
        scratch_types=[
            pltpu.VMEM((_C,), jnp.int32),
            pltpu.VMEM((_C,), jnp.int32),
            pltpu.VMEM((_C, 128), jnp.float32),
            pltpu.VMEM((_C, 128), jnp.float32),
            pltpu.VMEM((_C,), jnp.float32),
            pltpu.VMEM((_C,), jnp.float32),
            pltpu.VMEM((_C, _LN), jnp.float32),
            pltpu.VMEM((_C,), jnp.float32),
            pltpu.SemaphoreType.DMA,
            pltpu.SemaphoreType.DMA,
            pltpu.SemaphoreType.DMA,
            pltpu.SemaphoreType.DMA,
        ],
    )
    return f(g, nrm, srcp, dstp)


def _row_spec():
    return pl.BlockSpec((_BN, 128), lambda i: (i, 0))


def _full_spec(shape):
    nd = len(shape)
    return pl.BlockSpec(shape, lambda i: (0,) * nd)


def _tc_call(body, grid, in_specs, out_specs, out_shapes):
    return pl.pallas_call(
        body,
        grid=(grid,),
        in_specs=in_specs,
        out_specs=out_specs,
        out_shape=out_shapes,
    )


def _k_proj(x_ref, w_ref, b_ref, o_ref):
    o_ref[...] = (jnp.dot(x_ref[...], w_ref[...],
                          preferred_element_type=jnp.float32) + b_ref[...])


def _k_start(h_ref, w_ref, b_ref, t_ref, hid_ref, hh_ref):
    h = h_ref[...]
    hid_ref[...] = h * t_ref[...]
    hh_ref[...] = (jnp.dot(h, w_ref[...],
                           preferred_element_type=jnp.float32) + b_ref[...])


def _k_step(p_ref, hid_ref, w_ref, b_ref, t_ref, hido_ref, hh_ref):
    cur = jnp.maximum(p_ref[0] + p_ref[1], 0.0)
    hido_ref[...] = hid_ref[...] + cur * t_ref[...]
    hh_ref[...] = (jnp.dot(cur, w_ref[...],
                           preferred_element_type=jnp.float32) + b_ref[...])


def _k_fin(p_ref, hid_ref, t_ref, o_ref):
    cur = jnp.maximum(p_ref[0] + p_ref[1], 0.0)
    o_ref[...] = hid_ref[...] + cur * t_ref[...]


def _k_fe(h_ref, w1_ref, b1_ref, w2_ref, b2_ref, g_ref, n_ref):
    z = (jnp.dot(h_ref[...], w1_ref[...],
                 preferred_element_type=jnp.float32) + b1_ref[...])
    z = jnp.where(z > 0, z, jnp.exp(z) - 1.0)
    g = (jnp.dot(z, w2_ref[...],
                 preferred_element_type=jnp.float32) + b2_ref[...])
    g_ref[...] = g
    n_ref[...] = jnp.sqrt(jnp.sum(g * g, axis=1, keepdims=True))


def _group_mats():
    k = lax.broadcasted_iota(jnp.int32, (128, 8), 0)
    j = lax.broadcasted_iota(jnp.int32, (128, 8), 1)
    s = jnp.where(k // _LN == j, 1.0, 0.0).astype(jnp.float32)
    return s, s.T


def _k_expand(w8_ref, o_ref):
    _, st = _group_mats()
    o_ref[...] = jnp.dot(w8_ref[...], st, preferred_element_type=jnp.float32)


def _k_attn_fin(pb_ref, den8_ref, w8_ref, o_ref):
    s, st = _group_mats()
    num8 = jnp.dot(pb_ref[...], s, preferred_element_type=jnp.float32)
    wa8 = w8_ref[...] * num8 / den8_ref[...]
    o_ref[...] = jnp.dot(wa8, st, preferred_element_type=jnp.float32)


def kernel(x, edge_index, edge_weight, W_in, b_in, convW, convB, temp,
           feW1, feB1, feW2, feB2, W_out, b_out):
    n, d = x.shape
    e = edge_index.shape[1]
    nl = convW.shape[0]

    ch = -(-e // (_NC * _NS * _C))
    epad = _NC * _NS * ch * _C
    pad = epad - e
    eg = epad // 8

    src = jnp.concatenate([edge_index[0], jnp.zeros((pad,), jnp.int32)])
    dst = jnp.concatenate([edge_index[1], jnp.zeros((pad,), jnp.int32)])
    wp = jnp.concatenate([edge_weight, jnp.zeros((pad,), jnp.float32)])
    zeros = jnp.zeros((n, d), jnp.float32)

    tb = temp[:, None] * jnp.ones((1, d), jnp.float32)
    b2 = lambda b: b.reshape(1, d)

    row = _row_spec()
    wsp = _full_spec((d, d))
    bsp = _full_spec((1, d))
    psp = pl.BlockSpec((_NC, _BN, d), lambda i: (0, i, 0))
    nsp = pl.BlockSpec((_BN, 1), lambda i: (i, 0))
    erow = pl.BlockSpec((_BE, 128), lambda i: (i, 0))
    erow8 = pl.BlockSpec((_BE, 8), lambda i: (i, 0))
    sh = jax.ShapeDtypeStruct((n, d), jnp.float32)
    she = jax.ShapeDtypeStruct((eg, 128), jnp.float32)

    gn = n // _BN
    ge = eg // _BE

    h = _tc_call(_k_proj, gn, [row, wsp, bsp], row, sh)(x, W_in.T, b2(b_in))

    wexp1 = _tc_call(_k_expand, ge, [erow8], erow, she)(
        wp.reshape(eg, 8)).reshape(epad, _LN)

    def gpr(wexp):
        hid, hh = _tc_call(_k_start, gn, [row, wsp, bsp, bsp], [row, row],
                           [sh, sh])(h, convW[0].T, b2(convB[0]), tb[0:1])
        for i in range(nl):
            p = _edge_pass(hh, src, dst, wexp, zeros, n, ch)
            if i + 1 < nl:
                hid, hh = _tc_call(
                    _k_step, gn, [psp, row, wsp, bsp, bsp], [row, row],
                    [sh, sh])(p, hid, convW[i + 1].T, b2(convB[i + 1]),
                              tb[i + 1:i + 2])
            else:
                hid = _tc_call(_k_fin, gn, [psp, row, bsp], row, sh)(
                    p, hid, tb[i + 1:i + 2])
        return hid

    h_gnn = gpr(wexp1)

    g, nrm = _tc_call(_k_fe, gn, [row, wsp, bsp, wsp, bsp], [row, nsp],
                      [sh, jax.ShapeDtypeStruct((n, 1), jnp.float32)])(
        h_gnn, feW1.T, b2(feB1), feW2.T, b2(feB2))

    pbuf, den = _edge_attn(g, nrm.reshape(n), src, dst, epad, ch)

    wexp2 = _tc_call(_k_attn_fin, ge, [erow, erow8, erow8], erow, she)(
        pbuf.reshape(eg, 128), den.reshape(eg, 8),
        wp.reshape(eg, 8)).reshape(epad, _LN)

    h_gnn2 = gpr(wexp2)

    out = _tc_call(_k_proj, gn, [row, wsp, bsp], row, sh)(
        h_gnn2, W_out.T, b2(b_out))
    return out
